# baseline (speedup 1.0000x reference)
import jax
import jax.numpy as jnp
from jax import lax
from jax.experimental import pallas as pl
from jax.experimental.pallas import tpu as pltpu
from jax.experimental.pallas import tpu_sc as plsc

VOCAB = 1000000
D = 64
SEQ = 200
B = 4096
DP = 128

NC, NS, L = 2, 16, 16
NW = NC * NS
NTC = B // DP
NBLK = SEQ * NTC
BPT = NBLK // NW
NBUF = 2


def _body(idx_hbm, table_hbm, pos_hbm, out_hbm,
          idx_v, rows, outb, pos_v, gsem0, gsem1, osem0, osem1):
    c = lax.axis_index("c")
    s_ax = lax.axis_index("s")
    wid = s_ax * NC + c
    blk0 = pl.multiple_of(wid * BPT, 8)

    pltpu.sync_copy(pos_hbm, pos_v)
    pltpu.sync_copy(idx_hbm.at[pl.ds(blk0, BPT)], idx_v)

    gsems = (gsem0, gsem1)
    osems = (osem0, osem1)

    ridx = [lax.iota(jnp.int32, L) + c16 * L for c16 in range(DP // L)]

    def gather_desc(k, b, sem):
        return pltpu.make_async_copy(table_hbm.at[idx_v.at[k]], rows.at[b], sem)

    def out_descs(k, b, sem):
        gk = blk0 + k
        s = gk // NTC
        tc = gk - s * NTC
        return [pltpu.make_async_copy(outb.at[b, tr], out_hbm.at[s, tr, tc], sem)
                for tr in range(D // 8)]

    for b in range(NBUF):
        gather_desc(b, b, gsems[b]).start()

    @pl.loop(0, BPT, step=NBUF)
    def blk(kk):
        for b in range(NBUF):
            k = kk + b
            gk = blk0 + k
            s = gk // NTC
            gather_desc(k, b, gsems[b]).wait()

            @pl.when(k >= NBUF)
            def _():
                for d_ in out_descs(k - NBUF, b, osems[b]):
                    d_.wait()

            @pl.loop(0, D, unroll=8)
            def dloop(d):
                pidx = jnp.full((L,), s * D + d, jnp.int32)
                p = plsc.load_gather(pos_v, [pidx])
                cidx = jnp.full((L,), d, jnp.int32)
                tr = d // 8
                r = d - tr * 8
                for c16 in range(DP // L):
                    g = plsc.load_gather(rows.at[b], [ridx[c16], cidx])
                    outb[b, tr, r, pl.ds(c16 * L, L)] = g + p

            @pl.when(k + NBUF < BPT)
            def _():
                gather_desc(k + NBUF, b, gsems[b]).start()

            for d_ in out_descs(k, b, osems[b]):
                d_.start()

    for b in range(NBUF):
        for d_ in out_descs(BPT - NBUF + b, b, osems[b]):
            d_.wait()


@jax.jit
def kernel(input_idx, word_table, pos_table):
    idxT = input_idx.T.reshape(NBLK, DP).astype(jnp.int32)
    tablep = jnp.pad(word_table, ((0, 0), (0, DP - D)))
    pos_flat = pos_table.reshape(-1)
    mesh = plsc.VectorSubcoreMesh(core_axis_name="c", subcore_axis_name="s")
    out5 = pl.kernel(
        _body,
        out_type=jax.ShapeDtypeStruct((SEQ, D // 8, NTC, 8, DP), jnp.float32),
        mesh=mesh,
        compiler_params=pltpu.CompilerParams(use_tc_tiling_on_sc=False, needs_layout_passes=False),
        scratch_types=[
            pltpu.VMEM((BPT, DP), jnp.int32),
            pltpu.VMEM((NBUF, DP, DP), jnp.float32),
            pltpu.VMEM((NBUF, D // 8, 8, DP), jnp.float32),
            pltpu.VMEM((SEQ * D,), jnp.float32),
            pltpu.SemaphoreType.DMA,
            pltpu.SemaphoreType.DMA,
            pltpu.SemaphoreType.DMA,
            pltpu.SemaphoreType.DMA,
        ],
    )(idxT, tablep, pos_flat)
    return out5.transpose(2, 4, 0, 1, 3).reshape(B, SEQ, D)


# speedup vs baseline: 1.7432x; 1.7432x over previous
import jax
import jax.numpy as jnp
from jax import lax
from jax.experimental import pallas as pl
from jax.experimental.pallas import tpu as pltpu
from jax.experimental.pallas import tpu_sc as plsc

VOCAB = 1000000
D = 64
SEQ = 200
B = 4096
DP = 128

NC, NS, L = 2, 16, 16
NW = NC * NS
N = B * SEQ
ROWS_PER_W = N // NW
G = 128
CR = 128
NSTREAM = CR // G
CHUNKS = ROWS_PER_W // CR
NBUF = 4
IBLKS = ROWS_PER_W // G


def _body(idx_hbm, table_hbm, pos_hbm, out_hbm,
          idx_v, rows, pos_v, gsem0, gsem1, gsem2, gsem3, osem0, osem1, osem2, osem3):
    c = lax.axis_index("c")
    s = lax.axis_index("s")
    wid = s * NC + c
    base = wid * ROWS_PER_W
    iblk = pl.multiple_of(wid * IBLKS, 8)

    pltpu.sync_copy(pos_hbm, pos_v)
    pltpu.sync_copy(idx_hbm.at[pl.ds(iblk, IBLKS)], idx_v)

    gsems = (gsem0, gsem1, gsem2, gsem3)
    osems = (osem0, osem1, osem2, osem3)

    def gather_descs(gg, b, sem):
        return [pltpu.make_async_copy(
                    table_hbm.at[idx_v.at[gg * NSTREAM + j]],
                    rows.at[b, pl.ds(j * G, G)], sem)
                for j in range(NSTREAM)]

    def out_desc(gg, b, sem):
        r0 = pl.multiple_of(base + gg * CR, 8)
        return pltpu.make_async_copy(rows.at[b], out_hbm.at[pl.ds(r0, CR)], sem)

    for b in range(NBUF):
        for d_ in gather_descs(b, b, gsems[b]):
            d_.start()

    @pl.loop(0, CHUNKS, step=NBUF)
    def chunk(g):
        for b in range(NBUF):
            gg = g + b
            for d_ in gather_descs(gg, b, gsems[b]):
                d_.wait()

            @pl.when(gg >= NBUF)
            def _():
                out_desc(gg - NBUF, b, osems[b]).wait()

            p0 = lax.rem(gg * CR, SEQ)

            @pl.loop(0, CR)
            def posrow(k):
                p = p0 + k
                p = lax.select(p >= 2 * SEQ, p - 2 * SEQ,
                               lax.select(p >= SEQ, p - SEQ, p))
                for d2 in range(D // L):
                    v = pos_v[pl.ds(p * D + d2 * L, L)]
                    plsc.addupdate(rows.at[b, k, pl.ds(d2 * L, L)], v)

            @pl.when(gg + NBUF < CHUNKS)
            def _():
                for d_ in gather_descs(gg + NBUF, b, gsems[b]):
                    d_.start()

            out_desc(gg, b, osems[b]).start()

    for b in range(NBUF):
        out_desc(CHUNKS - NBUF + b, b, osems[b]).wait()


@jax.jit
def kernel(input_idx, word_table, pos_table):
    idx2 = input_idx.reshape(N // G, G).astype(jnp.int32)
    tablep = jnp.pad(word_table, ((0, 0), (0, DP - D)))
    pos_flat = pos_table.reshape(-1)
    mesh = plsc.VectorSubcoreMesh(core_axis_name="c", subcore_axis_name="s")
    out = pl.kernel(
        _body,
        out_type=jax.ShapeDtypeStruct((N, DP), jnp.float32),
        mesh=mesh,
        compiler_params=pltpu.CompilerParams(use_tc_tiling_on_sc=False),
        scratch_types=[
            pltpu.VMEM((IBLKS, G), jnp.int32),
            pltpu.VMEM((NBUF, CR, DP), jnp.float32),
            pltpu.VMEM((SEQ * D,), jnp.float32),
            pltpu.SemaphoreType.DMA,
            pltpu.SemaphoreType.DMA,
            pltpu.SemaphoreType.DMA,
            pltpu.SemaphoreType.DMA,
            pltpu.SemaphoreType.DMA,
            pltpu.SemaphoreType.DMA,
            pltpu.SemaphoreType.DMA,
            pltpu.SemaphoreType.DMA,
        ],
    )(idx2, tablep, pos_flat)
    return out[:, :D].reshape(B, SEQ, D)


# speedup vs baseline: 1.7478x; 1.0026x over previous
import jax
import jax.numpy as jnp
from jax import lax
from jax.experimental import pallas as pl
from jax.experimental.pallas import tpu as pltpu
from jax.experimental.pallas import tpu_sc as plsc

VOCAB = 1000000
D = 64
SEQ = 200
B = 4096
DP = 128

NC, NS, L = 2, 16, 16
NW = NC * NS
N = B * SEQ
ROWS_PER_W = N // NW
G = 128
CR = 256
NSTREAM = CR // G
CHUNKS = ROWS_PER_W // CR
NBUF = 2
IBLKS = ROWS_PER_W // G


def _body(idx_hbm, table_hbm, pos_hbm, out_hbm,
          idx_v, rows, pos_v, gsem0, gsem1, osem0, osem1):
    c = lax.axis_index("c")
    s = lax.axis_index("s")
    wid = s * NC + c
    base = wid * ROWS_PER_W
    iblk = pl.multiple_of(wid * IBLKS, 8)

    pltpu.sync_copy(pos_hbm, pos_v)
    pltpu.sync_copy(idx_hbm.at[pl.ds(iblk, IBLKS)], idx_v)

    gsems = (gsem0, gsem1)
    osems = (osem0, osem1)

    def gather_descs(gg, b, sem):
        return [pltpu.make_async_copy(
                    table_hbm.at[idx_v.at[gg * NSTREAM + j]],
                    rows.at[b, pl.ds(j * G, G)], sem)
                for j in range(NSTREAM)]

    def out_desc(gg, b, sem):
        r0 = pl.multiple_of(base + gg * CR, 8)
        return pltpu.make_async_copy(rows.at[b], out_hbm.at[pl.ds(r0, CR)], sem)

    for b in range(NBUF):
        for d_ in gather_descs(b, b, gsems[b]):
            d_.start()

    @pl.loop(0, CHUNKS, step=NBUF)
    def chunk(g):
        for b in range(NBUF):
            gg = g + b
            for d_ in gather_descs(gg, b, gsems[b]):
                d_.wait()

            @pl.when(gg >= NBUF)
            def _():
                out_desc(gg - NBUF, b, osems[b]).wait()

            p0 = lax.rem(gg * CR, SEQ)

            @pl.loop(0, CR)
            def posrow(k):
                p = p0 + k
                p = lax.select(p >= 2 * SEQ, p - 2 * SEQ,
                               lax.select(p >= SEQ, p - SEQ, p))
                for d2 in range(D // L):
                    v = pos_v[pl.ds(p * D + d2 * L, L)]
                    plsc.addupdate(rows.at[b, k, pl.ds(d2 * L, L)], v)

            @pl.when(gg + NBUF < CHUNKS)
            def _():
                for d_ in gather_descs(gg + NBUF, b, gsems[b]):
                    d_.start()

            out_desc(gg, b, osems[b]).start()

    for b in range(NBUF):
        out_desc(CHUNKS - NBUF + b, b, osems[b]).wait()


@jax.jit
def kernel(input_idx, word_table, pos_table):
    idx2 = input_idx.reshape(N // G, G).astype(jnp.int32)
    tablep = jnp.pad(word_table, ((0, 0), (0, DP - D)))
    pos_flat = pos_table.reshape(-1)
    mesh = plsc.VectorSubcoreMesh(core_axis_name="c", subcore_axis_name="s")
    out = pl.kernel(
        _body,
        out_type=jax.ShapeDtypeStruct((N, DP), jnp.float32),
        mesh=mesh,
        compiler_params=pltpu.CompilerParams(use_tc_tiling_on_sc=False),
        scratch_types=[
            pltpu.VMEM((IBLKS, G), jnp.int32),
            pltpu.VMEM((NBUF, CR, DP), jnp.float32),
            pltpu.VMEM((SEQ * D,), jnp.float32),
            pltpu.SemaphoreType.DMA,
            pltpu.SemaphoreType.DMA,
            pltpu.SemaphoreType.DMA,
            pltpu.SemaphoreType.DMA,
        ],
    )(idx2, tablep, pos_flat)
    return out[:, :D].reshape(B, SEQ, D)
